# baseline (speedup 1.0000x reference)
import functools

import jax
import jax.numpy as jnp
from jax import lax
from jax.experimental import pallas as pl
from jax.experimental.pallas import tpu as pltpu
from jax.experimental.pallas import tpu_sc as plsc

X_SHAPE = 1000
Y_SHAPE = 100000

NW = 32
H_BLK = 2
BI = 128


@functools.lru_cache(maxsize=None)
def _build(nb, hist, vocab, dim):
    nbt = nb // BI
    ch = H_BLK * BI
    nblk = hist // H_BLK
    dt_n = dim // 8
    tile = 8 * BI
    assert nbt == NW and hist % (2 * H_BLK) == 0 and dim % 16 == 0

    mesh = plsc.VectorSubcoreMesh(core_axis_name="c", subcore_axis_name="s")

    @functools.partial(
        pl.kernel,
        mesh=mesh,
        compiler_params=pltpu.CompilerParams(
            use_tc_tiling_on_sc=False, needs_layout_passes=False,
            disable_bounds_checks=True),
        out_type=jax.ShapeDtypeStruct((hist, dt_n, nbt, tile), jnp.float32),
        scratch_types=[
            pltpu.VMEM((hist, BI), jnp.int32),
            pltpu.VMEM((ch,), jnp.int32),
            pltpu.VMEM((ch,), jnp.int32),
            pltpu.VMEM((ch,), jnp.int32),
            pltpu.VMEM((ch,), jnp.int32),
            pltpu.VMEM((ch, dim), jnp.float32),
            pltpu.VMEM((ch, dim), jnp.float32),
            pltpu.VMEM((H_BLK * dt_n * tile,), jnp.float32),
            pltpu.VMEM((H_BLK * dt_n * tile,), jnp.float32),
            pltpu.SemaphoreType.DMA,
            pltpu.SemaphoreType.DMA,
            pltpu.SemaphoreType.DMA,
        ],
    )
    def k(pid_hbm, table_hbm, out_hbm, pid_slab,
          row_v0, row_v1, col_v0, col_v1, buf0, buf1, tb0, tb1,
          gsB, gsC, osem):
        wid = lax.axis_index("s") * 2 + lax.axis_index("c")
        row_v = (row_v0, row_v1)
        col_v = (col_v0, col_v1)
        buf = (buf0, buf1)
        tbuf = (tb0, tb1)
        eight = jnp.full((16,), 8, jnp.int32)
        lane = lax.iota(jnp.int32, 16)

        def div_approx(p, d):
            q = (p.astype(jnp.float32) * jnp.float32(1.0 / d)).astype(jnp.int32)
            r = p - q * d
            q = q + jnp.where(r >= d, 1, 0) - jnp.where(r < 0, 1, 0)
            return q
        fvec = [lax.div(cc * 16 + lane, eight) * tile
                + lax.rem(cc * 16 + lane, eight) * BI
                for cc in range(dim // 16)]

        def convload(kb, s):
            @plsc.parallel_loop(0, ch // 16, unroll=4)
            def body(i):
                hh = i // (BI // 16)
                ii = i - hh * (BI // 16)
                p = pid_slab[kb * H_BLK + hh, pl.ds(ii * 16, 16)]
                sl = pl.ds(i * 16, 16)
                row_v[s][sl] = div_approx(p, X_SHAPE)
                col_v[s][sl] = p - div_approx(p, Y_SHAPE) * Y_SHAPE

        def fire_b(s):
            for j in range(H_BLK):
                pltpu.async_copy(
                    table_hbm.at[row_v[s].at[pl.ds(j * BI, BI)]],
                    buf[s].at[pl.ds(j * BI, BI)], gsB)

        def wait_b(s):
            for j in range(H_BLK):
                pltpu.make_async_copy(
                    table_hbm.at[row_v[s].at[pl.ds(j * BI, BI)]],
                    buf[s].at[pl.ds(j * BI, BI)], gsB).wait()

        def fire_c(s):
            for j in range(H_BLK):
                pltpu.async_copy(
                    table_hbm.at[col_v[s].at[pl.ds(j * BI, BI)]],
                    buf[s].at[pl.ds(j * BI, BI)], gsC, add=True)

        def wait_c(s):
            for j in range(H_BLK):
                pltpu.make_async_copy(
                    table_hbm.at[col_v[s].at[pl.ds(j * BI, BI)]],
                    buf[s].at[pl.ds(j * BI, BI)], gsC).wait()

        def transpose(s):
            @plsc.parallel_loop(0, ch, unroll=4)
            def body(l):
                hh = l // BI
                bi = l - hh * BI
                base = jnp.full((16,), hh * (dt_n * tile) + bi, jnp.int32)
                for cc in range(dim // 16):
                    v = buf[s][l, pl.ds(cc * 16, 16)]
                    plsc.store_scatter(tbuf[s], [fvec[cc] + base], v)

        def fire_d(kb, s):
            for hh in range(H_BLK):
                for dt in range(dt_n):
                    pltpu.async_copy(
                        tbuf[s].at[pl.ds((hh * dt_n + dt) * tile, tile)],
                        out_hbm.at[kb * H_BLK + hh, dt, wid], osem)

        def wait_d(s):
            for _ in range(H_BLK * dt_n):
                pltpu.make_async_copy(
                    tbuf[s].at[pl.ds(0, tile)],
                    out_hbm.at[0, 0, wid], osem).wait()

        pltpu.sync_copy(pid_hbm.at[:, pl.ds(wid * BI, BI)], pid_slab)

        convload(0, 0)
        fire_b(0)
        wait_b(0)
        fire_c(0)
        convload(1, 1)
        fire_b(1)
        wait_c(0)
        transpose(0)
        fire_d(0, 0)
        wait_b(1)
        fire_c(1)
        convload(2, 0)
        fire_b(0)
        wait_c(1)
        transpose(1)
        fire_d(1, 1)

        def step(kb_next, s):
            wait_b(s)
            fire_c(s)
            o = 1 - s
            convload(kb_next, o)
            fire_b(o)
            wait_c(s)
            wait_d(s)
            transpose(s)

        def pair(p, carry):
            k0 = 2 * p
            step(k0 + 1, 0)
            fire_d(k0, 0)
            step(k0 + 2, 1)
            fire_d(k0 + 1, 1)
            return carry

        lax.fori_loop(1, nblk // 2 - 1, pair, 0)

        last = nblk - 1
        step(last, 0)
        fire_d(last - 1, 0)
        wait_b(1)
        fire_c(1)
        wait_c(1)
        wait_d(1)
        transpose(1)
        fire_d(last, 1)
        wait_d(0)
        wait_d(1)

    return k


def kernel(position_ids, y_table):
    nb, hist = position_ids.shape
    vocab, dim = y_table.shape
    pid_t = position_ids.T
    o4 = _build(nb, hist, vocab, dim)(pid_t, y_table)
    o5 = o4.reshape(hist, dim // 8, nb // 128, 8, 128)
    return o5.transpose((2, 4, 0, 1, 3)).reshape(nb, hist, dim)


# speedup vs baseline: 1.8530x; 1.8530x over previous
import functools

import jax
import jax.numpy as jnp
from jax import lax
from jax.experimental import pallas as pl
from jax.experimental.pallas import tpu as pltpu
from jax.experimental.pallas import tpu_sc as plsc

X_SHAPE = 1000
Y_SHAPE = 100000

NW = 32
H_BLK = 2
BI = 128


@functools.lru_cache(maxsize=None)
def _build(nb, hist, vocab, dim):
    nbt = nb // BI
    ch = H_BLK * BI
    nblk = hist // H_BLK
    dt_n = dim // 8
    tile = 8 * BI
    assert nbt == NW and hist % (2 * H_BLK) == 0 and dim % 16 == 0

    mesh = plsc.VectorSubcoreMesh(core_axis_name="c", subcore_axis_name="s")

    @functools.partial(
        pl.kernel,
        mesh=mesh,
        compiler_params=pltpu.CompilerParams(
            use_tc_tiling_on_sc=False, needs_layout_passes=False,
            disable_bounds_checks=True),
        out_type=jax.ShapeDtypeStruct((hist, dt_n, nbt, tile), jnp.float32),
        scratch_types=[
            pltpu.VMEM((hist, BI), jnp.int32),
            pltpu.VMEM((ch,), jnp.int32),
            pltpu.VMEM((ch,), jnp.int32),
            pltpu.VMEM((ch,), jnp.int32),
            pltpu.VMEM((ch,), jnp.int32),
            pltpu.VMEM((ch, dim), jnp.float32),
            pltpu.VMEM((ch, dim), jnp.float32),
            pltpu.VMEM((H_BLK * dt_n * tile,), jnp.float32),
            pltpu.VMEM((H_BLK * dt_n * tile,), jnp.float32),
            pltpu.SemaphoreType.DMA,
            pltpu.SemaphoreType.DMA,
            pltpu.SemaphoreType.DMA,
        ],
    )
    def k(pid_hbm, table_hbm, out_hbm, pid_slab,
          row_v0, row_v1, col_v0, col_v1, buf0, buf1, tb0, tb1,
          gsB, gsC, osem):
        wid = lax.axis_index("s") * 2 + lax.axis_index("c")
        row_v = (row_v0, row_v1)
        col_v = (col_v0, col_v1)
        buf = (buf0, buf1)
        tbuf = (tb0, tb1)
        lane = lax.iota(jnp.int32, 16)
        mvec = [lax.rem(lane + k, jnp.full((16,), 16, jnp.int32))
                for k in range(16)]
        svec = [m * BI + lane for m in mvec]

        def div_approx(p, d):
            q = (p.astype(jnp.float32) * jnp.float32(1.0 / d)).astype(jnp.int32)
            r = p - q * d
            q = q + jnp.where(r >= d, 1, 0) - jnp.where(r < 0, 1, 0)
            return q

        def convload(kb, s):
            @plsc.parallel_loop(0, ch // 16, unroll=4)
            def body(i):
                hh = i // (BI // 16)
                ii = i - hh * (BI // 16)
                p = pid_slab[kb * H_BLK + hh, pl.ds(ii * 16, 16)]
                sl = pl.ds(i * 16, 16)
                row_v[s][sl] = div_approx(p, X_SHAPE)
                col_v[s][sl] = p - div_approx(p, Y_SHAPE) * Y_SHAPE

        def fire_b(s):
            for j in range(H_BLK):
                pltpu.async_copy(
                    table_hbm.at[row_v[s].at[pl.ds(j * BI, BI)]],
                    buf[s].at[pl.ds(j * BI, BI)], gsB)

        def wait_b(s):
            for j in range(H_BLK):
                pltpu.make_async_copy(
                    table_hbm.at[row_v[s].at[pl.ds(j * BI, BI)]],
                    buf[s].at[pl.ds(j * BI, BI)], gsB).wait()

        def fire_c(s):
            for j in range(H_BLK):
                pltpu.async_copy(
                    table_hbm.at[col_v[s].at[pl.ds(j * BI, BI)]],
                    buf[s].at[pl.ds(j * BI, BI)], gsC, add=True)

        def wait_c(s):
            for j in range(H_BLK):
                pltpu.make_async_copy(
                    table_hbm.at[col_v[s].at[pl.ds(j * BI, BI)]],
                    buf[s].at[pl.ds(j * BI, BI)], gsC).wait()

        def transpose(s):
            n_l0 = BI // 16
            n_d0 = dim // 16

            @plsc.parallel_loop(0, H_BLK * n_l0 * n_d0, unroll=2)
            def body(t):
                hh = t // (n_l0 * n_d0)
                rest = t - hh * (n_l0 * n_d0)
                l0 = (rest // n_d0) * 16
                d0 = (rest - (rest // n_d0) * n_d0) * 16
                lvec = lane + jnp.full((16,), hh * BI + l0, jnp.int32)
                sbase = jnp.full((16,), hh * (dt_n * tile) + d0 * (tile // 8) + l0,
                                 jnp.int32)
                for k in range(16):
                    v = plsc.load_gather(buf[s], [lvec, mvec[k] + d0])
                    plsc.store_scatter(tbuf[s], [svec[k] + sbase], v)

        def fire_d(kb, s):
            for hh in range(H_BLK):
                for dt in range(dt_n):
                    pltpu.async_copy(
                        tbuf[s].at[pl.ds((hh * dt_n + dt) * tile, tile)],
                        out_hbm.at[kb * H_BLK + hh, dt, wid], osem)

        def wait_d(s):
            for _ in range(H_BLK * dt_n):
                pltpu.make_async_copy(
                    tbuf[s].at[pl.ds(0, tile)],
                    out_hbm.at[0, 0, wid], osem).wait()

        pltpu.sync_copy(pid_hbm.at[:, pl.ds(wid * BI, BI)], pid_slab)

        convload(0, 0)
        fire_b(0)
        wait_b(0)
        fire_c(0)
        convload(1, 1)
        fire_b(1)
        wait_c(0)
        transpose(0)
        fire_d(0, 0)
        wait_b(1)
        fire_c(1)
        convload(2, 0)
        fire_b(0)
        wait_c(1)
        transpose(1)
        fire_d(1, 1)

        def step(kb_next, s):
            wait_b(s)
            fire_c(s)
            o = 1 - s
            convload(kb_next, o)
            fire_b(o)
            wait_c(s)
            wait_d(s)
            transpose(s)

        def pair(p, carry):
            k0 = 2 * p
            step(k0 + 1, 0)
            fire_d(k0, 0)
            step(k0 + 2, 1)
            fire_d(k0 + 1, 1)
            return carry

        lax.fori_loop(1, nblk // 2 - 1, pair, 0)

        last = nblk - 1
        step(last, 0)
        fire_d(last - 1, 0)
        wait_b(1)
        fire_c(1)
        wait_c(1)
        wait_d(1)
        transpose(1)
        fire_d(last, 1)
        wait_d(0)
        wait_d(1)

    return k


def kernel(position_ids, y_table):
    nb, hist = position_ids.shape
    vocab, dim = y_table.shape
    pid_t = position_ids.T
    o4 = _build(nb, hist, vocab, dim)(pid_t, y_table)
    o5 = o4.reshape(hist, dim // 8, nb // 128, 8, 128)
    return o5.transpose((2, 4, 0, 1, 3)).reshape(nb, hist, dim)


# speedup vs baseline: 2.7467x; 1.4823x over previous
import functools

import jax
import jax.numpy as jnp
from jax import lax
from jax.experimental import pallas as pl
from jax.experimental.pallas import tpu as pltpu
from jax.experimental.pallas import tpu_sc as plsc

X_SHAPE = 1000
Y_SHAPE = 100000

NW = 32
H_BLK = 2
BI = 128


@functools.lru_cache(maxsize=None)
def _build(nb, hist, vocab, dim):
    nbt = nb // BI
    ch = H_BLK * BI
    nblk = hist // H_BLK
    dt_n = dim // 8
    tile = 8 * BI
    assert nbt == NW and hist % (2 * H_BLK) == 0 and dim % 16 == 0

    mesh = plsc.VectorSubcoreMesh(core_axis_name="c", subcore_axis_name="s")

    @functools.partial(
        pl.kernel,
        mesh=mesh,
        compiler_params=pltpu.CompilerParams(
            use_tc_tiling_on_sc=False, needs_layout_passes=False,
            disable_bounds_checks=True),
        out_type=jax.ShapeDtypeStruct((hist, dt_n, nbt, tile), jnp.float32),
        scratch_types=[
            pltpu.VMEM((hist, BI), jnp.int32),
            pltpu.VMEM((ch,), jnp.int32),
            pltpu.VMEM((ch,), jnp.int32),
            pltpu.VMEM((ch,), jnp.int32),
            pltpu.VMEM((ch,), jnp.int32),
            pltpu.VMEM((ch, dim), jnp.float32),
            pltpu.VMEM((ch, dim), jnp.float32),
            pltpu.VMEM((H_BLK * dt_n * tile,), jnp.float32),
            pltpu.VMEM((H_BLK * dt_n * tile,), jnp.float32),
            pltpu.SemaphoreType.DMA,
            pltpu.SemaphoreType.DMA,
            pltpu.SemaphoreType.DMA,
        ],
    )
    def k(pid_hbm, table_hbm, out_hbm, pid_slab,
          row_v0, row_v1, col_v0, col_v1, buf0, buf1, tb0, tb1,
          gsB, gsC, osem):
        wid = lax.axis_index("s") * 2 + lax.axis_index("c")
        row_v = (row_v0, row_v1)
        col_v = (col_v0, col_v1)
        buf = (buf0, buf1)
        tbuf = (tb0, tb1)
        lane = lax.iota(jnp.int32, 16)
        mvec = [lax.rem(lane + k, jnp.full((16,), 16, jnp.int32))
                for k in range(16)]
        svec = [m * BI + lane for m in mvec]

        def div_approx(p, d):
            q = (p.astype(jnp.float32) * jnp.float32(1.0 / d)).astype(jnp.int32)
            r = p - q * d
            q = q + jnp.where(r >= d, 1, 0) - jnp.where(r < 0, 1, 0)
            return q

        def convload(kb, s):
            @plsc.parallel_loop(0, ch // 16, unroll=4)
            def body(i):
                hh = i // (BI // 16)
                ii = i - hh * (BI // 16)
                p = pid_slab[kb * H_BLK + hh, pl.ds(ii * 16, 16)]
                sl = pl.ds(i * 16, 16)
                row_v[s][sl] = div_approx(p, X_SHAPE)
                col_v[s][sl] = p - div_approx(p, Y_SHAPE) * Y_SHAPE

        def fire_b(s):
            for j in range(H_BLK):
                pltpu.async_copy(
                    table_hbm.at[row_v[s].at[pl.ds(j * BI, BI)]],
                    buf[s].at[pl.ds(j * BI, BI)], gsB)

        def wait_b(s):
            for j in range(H_BLK):
                pltpu.make_async_copy(
                    table_hbm.at[row_v[s].at[pl.ds(j * BI, BI)]],
                    buf[s].at[pl.ds(j * BI, BI)], gsB).wait()

        def fire_c(s):
            for j in range(H_BLK):
                pltpu.async_copy(
                    table_hbm.at[col_v[s].at[pl.ds(j * BI, BI)]],
                    buf[s].at[pl.ds(j * BI, BI)], gsC, add=True)

        def wait_c(s):
            for j in range(H_BLK):
                pltpu.make_async_copy(
                    table_hbm.at[col_v[s].at[pl.ds(j * BI, BI)]],
                    buf[s].at[pl.ds(j * BI, BI)], gsC).wait()

        def transpose(s):
            n_l0 = BI // 16
            n_d0 = dim // 16

            @plsc.parallel_loop(0, H_BLK * n_l0 * n_d0, unroll=2)
            def body(t):
                hh = t // (n_l0 * n_d0)
                rest = t - hh * (n_l0 * n_d0)
                l0 = (rest // n_d0) * 16
                d0 = (rest - (rest // n_d0) * n_d0) * 16
                lvec = lane + jnp.full((16,), hh * BI + l0, jnp.int32)
                sbase = jnp.full((16,), hh * (dt_n * tile) + d0 * (tile // 8) + l0,
                                 jnp.int32)
                for k in range(16):
                    v = plsc.load_gather(buf[s], [lvec, mvec[k] + d0])
                    plsc.store_scatter(tbuf[s], [svec[k] + sbase], v)

        def fire_d(kb, s):
            for hh in range(H_BLK):
                for dt in range(dt_n):
                    pltpu.async_copy(
                        tbuf[s].at[pl.ds((hh * dt_n + dt) * tile, tile)],
                        out_hbm.at[kb * H_BLK + hh, dt, wid], osem)

        def wait_d(s):
            for _ in range(H_BLK * dt_n):
                pltpu.make_async_copy(
                    tbuf[s].at[pl.ds(0, tile)],
                    out_hbm.at[0, 0, wid], osem).wait()

        pltpu.sync_copy(pid_hbm.at[:, pl.ds(wid * BI, BI)], pid_slab)


        convload(0, 0)
        fire_b(0)
        fire_c(0)
        convload(1, 1)
        fire_b(1)
        fire_c(1)
        wait_b(0)
        wait_c(0)
        transpose(0)
        fire_d(0, 0)
        convload(2, 0)
        fire_b(0)
        fire_c(0)
        wait_b(1)
        wait_c(1)
        transpose(1)
        fire_d(1, 1)

        def step(kb_next, s):
            o = 1 - s
            convload(kb_next, o)
            fire_b(o)
            fire_c(o)
            wait_b(s)
            wait_c(s)
            wait_d(s)
            transpose(s)

        def pair(p, carry):
            k0 = 2 * p
            step(k0 + 1, 0)
            fire_d(k0, 0)
            step(k0 + 2, 1)
            fire_d(k0 + 1, 1)
            return carry

        lax.fori_loop(1, nblk // 2 - 1, pair, 0)

        last = nblk - 1
        step(last, 0)
        fire_d(last - 1, 0)
        wait_b(1)
        wait_c(1)
        wait_d(1)
        transpose(1)
        fire_d(last, 1)
        wait_d(0)
        wait_d(1)

    return k


def kernel(position_ids, y_table):
    nb, hist = position_ids.shape
    vocab, dim = y_table.shape
    pid_t = position_ids.T
    o4 = _build(nb, hist, vocab, dim)(pid_t, y_table)
    o5 = o4.reshape(hist, dim // 8, nb // 128, 8, 128)
    return o5.transpose((2, 4, 0, 1, 3)).reshape(nb, hist, dim)
